# baseline (speedup 1.0000x reference)
import functools

import jax
import jax.numpy as jnp
from jax import lax
from jax.experimental import pallas as pl
from jax.experimental.pallas import tpu as pltpu
from jax.experimental.pallas import tpu_sc as plsc

N_NODES = 10000
N_EDGES = 320000
NC = 2
NS = 16
NW = NC * NS
EPW = N_EDGES // NW
CHUNK = 80
NCHUNK = EPW // CHUNK
GRP = 16
NPAD = 10240
ROWS_PER_SUB = NPAD // NS
BN = 1000


def _proj_block(h, wq, bq, wk, bk, wv, bv, ws, bs, q_ref, kv_ref, s_ref,
                C, W):
    q_ref[...] = jnp.dot(h, wq[...], preferred_element_type=jnp.float32) + bq[...]
    kv_ref[:, :C] = jnp.dot(h, wk[...], preferred_element_type=jnp.float32) + bk[...]
    kv_ref[:, C:2 * C] = jnp.dot(h, wv[...], preferred_element_type=jnp.float32) + bv[...]
    tail = lax.broadcasted_iota(jnp.int32, (BN, W - C), 1)
    kv_ref[:, 2 * C:] = jnp.where(tail == 0, 1.0, 0.0).astype(jnp.float32)
    s_ref[...] = jnp.dot(h, ws[...], preferred_element_type=jnp.float32) + bs[...]


def _proj_outs(C, W):
    ospec = pl.BlockSpec((BN, C), lambda i: (i, 0))
    vspec = pl.BlockSpec((BN, C + W), lambda i: (i, 0))
    shapes = [jax.ShapeDtypeStruct((N_NODES, C), jnp.float32),
              jax.ShapeDtypeStruct((N_NODES, C + W), jnp.float32),
              jax.ShapeDtypeStruct((N_NODES, C), jnp.float32)]
    return [ospec, vspec, ospec], shapes


def _make_proj(din, C, W):
    def body(h_ref, wq, bq, wk, bk, wv, bv, ws, bs, q_ref, kv_ref, s_ref):
        _proj_block(h_ref[...], wq, bq, wk, bk, wv, bv, ws, bs,
                    q_ref, kv_ref, s_ref, C, W)

    wspec = pl.BlockSpec((din, C), lambda i: (0, 0))
    bspec = pl.BlockSpec((1, C), lambda i: (0, 0))
    out_specs, out_shape = _proj_outs(C, W)
    return pl.pallas_call(
        body,
        grid=(N_NODES // BN,),
        in_specs=[pl.BlockSpec((BN, din), lambda i: (i, 0)),
                  wspec, bspec, wspec, bspec, wspec, bspec, wspec, bspec],
        out_specs=out_specs,
        out_shape=out_shape,
    )


def _combine_block(p_ref, s_ref, C):
    tot = p_ref[0] + p_ref[1]
    den = tot[:, C:C + 1]
    agg = tot[:, :C] / (den + 1e-16)
    return agg + s_ref[...]


def _make_comb_proj(C, WIN, CO, WO):
    def body(p_ref, s_ref, g_ref, b_ref, wq, bq, wk, bk, wv, bv, ws, bs,
             q_ref, kv_ref, s2_ref):
        pre = _combine_block(p_ref, s_ref, C)
        mu = jnp.mean(pre, axis=-1, keepdims=True)
        var = jnp.mean((pre - mu) ** 2, axis=-1, keepdims=True)
        y = (pre - mu) / jnp.sqrt(var + 1e-5) * g_ref[...] + b_ref[...]
        h = jnp.where(y >= 0, y, 0.01 * y)
        _proj_block(h, wq, bq, wk, bk, wv, bv, ws, bs,
                    q_ref, kv_ref, s2_ref, CO, WO)

    wspec = pl.BlockSpec((C, CO), lambda i: (0, 0))
    bspec = pl.BlockSpec((1, CO), lambda i: (0, 0))
    out_specs, out_shape = _proj_outs(CO, WO)
    return pl.pallas_call(
        body,
        grid=(N_NODES // BN,),
        in_specs=[pl.BlockSpec((2, BN, WIN), lambda i: (0, i, 0)),
                  pl.BlockSpec((BN, C), lambda i: (i, 0)),
                  pl.BlockSpec((1, C), lambda i: (0, 0)),
                  pl.BlockSpec((1, C), lambda i: (0, 0)),
                  wspec, bspec, wspec, bspec, wspec, bspec, wspec, bspec],
        out_specs=out_specs,
        out_shape=out_shape,
    )


def _make_combine_final(C, WIN):
    def body(p_ref, s_ref, h_ref):
        pre = _combine_block(p_ref, s_ref, C)
        nrm = jnp.sqrt(jnp.sum(pre * pre, axis=-1, keepdims=True))
        y = pre / jnp.maximum(nrm, 1e-12)
        h_ref[...] = jnp.where(y >= 0, y, 0.01 * y)

    return pl.pallas_call(
        body,
        grid=(N_NODES // BN,),
        in_specs=[pl.BlockSpec((2, BN, WIN), lambda i: (0, i, 0)),
                  pl.BlockSpec((BN, C), lambda i: (i, 0))],
        out_specs=pl.BlockSpec((BN, C), lambda i: (i, 0)),
        out_shape=jax.ShapeDtypeStruct((N_NODES, C), jnp.float32),
    )


def _make_sc_attention(C, W):
    mesh = plsc.VectorSubcoreMesh(core_axis_name="c", subcore_axis_name="s",
                                  num_cores=NC, num_subcores=NS)
    inv = 1.0 / (C ** 0.5)
    KV = 2 * C + (W - C)

    @functools.partial(
        pl.kernel,
        out_type=jax.ShapeDtypeStruct((NC, NPAD, W), jnp.float32),
        mesh=mesh,
        compiler_params=pltpu.CompilerParams(needs_layout_passes=False,
                                             use_tc_tiling_on_sc=False),
        scratch_types=[
            pltpu.VMEM((2, CHUNK), jnp.int32),
            pltpu.VMEM((2, CHUNK), jnp.int32),
            pltpu.VMEM((CHUNK,), jnp.int32),
            pltpu.VMEM((CHUNK,), jnp.int32),
            pltpu.VMEM((CHUNK, C), jnp.float32),
            pltpu.VMEM((CHUNK, C), jnp.float32),
            pltpu.VMEM((CHUNK, KV), jnp.float32),
            pltpu.VMEM((CHUNK, KV), jnp.float32),
            pltpu.VMEM((CHUNK, W), jnp.float32),
            pltpu.VMEM((CHUNK, W), jnp.float32),
            pltpu.VMEM_SHARED((NPAD, W), jnp.float32),
            pltpu.SemaphoreType.DMA,
            pltpu.SemaphoreType.DMA,
            pltpu.SemaphoreType.DMA,
            pltpu.SemaphoreType.DMA,
            pltpu.SemaphoreType.DMA,
            pltpu.SemaphoreType.DMA,
        ],
    )
    def sc_att(q_hbm, kv_hbm, eidx_hbm, zeros_hbm, out_hbm,
               eidx0, eidx1, sdst0, sdst1,
               qr0, qr1, kvr0, kvr1, sr0, sr1, acc,
               sg0, sg1, si0, si1, ss0, ss1):
        eidx = (eidx0, eidx1)
        sdst = (sdst0, sdst1)
        qrs = (qr0, qr1)
        kvrs = (kvr0, kvr1)
        srs = (sr0, sr1)
        sg = (sg0, sg1)
        si = (si0, si1)
        ss = (ss0, ss1)
        cid = lax.axis_index("c")
        sid = lax.axis_index("s")
        wid = sid * NC + cid
        base = wid * NCHUNK

        def idx_issue(ch, b):
            pltpu.async_copy(eidx_hbm.at[base + ch], eidx[b], si[b])

        def idx_wait(b):
            pltpu.make_async_copy(eidx_hbm.at[base], eidx[b], si[b]).wait()

        def gath_issue(b):
            pltpu.async_copy(q_hbm.at[eidx[b].at[1]], qrs[b], sg[b])
            pltpu.async_copy(kv_hbm.at[eidx[b].at[0]], kvrs[b], sg[b])

        def gath_wait(b):
            pltpu.make_async_copy(q_hbm.at[eidx[b].at[1]], qrs[b], sg[b]).wait()
            pltpu.make_async_copy(kv_hbm.at[eidx[b].at[0]], kvrs[b], sg[b]).wait()

        def scat_issue(b):
            pltpu.async_copy(srs[b], acc.at[sdst[b]], ss[b], add=True)

        def scat_wait(b):
            pltpu.make_async_copy(srs[b], acc.at[sdst[b]], ss[b]).wait()

        def copy_sdst(b):
            for j in range(CHUNK // GRP):
                sdst[b][pl.ds(j * GRP, GRP)] = eidx[b][1, pl.ds(j * GRP, GRP)]

        ids = lax.iota(jnp.int32, GRP)

        def compute(b):
            evs = []
            for g in range(CHUNK // GRP):
                av = jnp.zeros((GRP,), jnp.float32)
                for i in range(GRP):
                    ei = g * GRP + i
                    d = jnp.zeros((GRP,), jnp.float32)
                    for h in range(C // GRP):
                        qv = qrs[b][ei, pl.ds(h * GRP, GRP)]
                        kv = kvrs[b][ei, pl.ds(h * GRP, GRP)]
                        d = d + qv * kv
                    av = jnp.where(ids == i, jnp.sum(d), av)
                evs.append(jnp.exp(av * inv))
            for g in range(CHUNK // GRP):
                ev = evs[g]
                for i in range(GRP):
                    ei = g * GRP + i
                    es = ev[i]
                    for h in range(W // GRP):
                        srs[b][ei, pl.ds(h * GRP, GRP)] = (
                            kvrs[b][ei, pl.ds(C + h * GRP, GRP)] * es)

        pltpu.sync_copy(zeros_hbm,
                        acc.at[pl.ds(sid * ROWS_PER_SUB, ROWS_PER_SUB)])
        plsc.subcore_barrier()

        idx_issue(0, 0)
        idx_wait(0)
        gath_issue(0)
        idx_issue(1, 1)

        def pair_body(j, carry):
            for t in range(2):
                ch = 2 * j + t
                b = t
                nb = 1 - t
                idx_wait(nb)
                gath_issue(nb)
                gath_wait(b)

                @pl.when(ch >= 1)
                def _():
                    scat_wait(nb)

                copy_sdst(b)

                @pl.when(ch + 2 < NCHUNK)
                def _():
                    idx_issue(ch + 2, b)

                compute(b)
                scat_issue(b)
            return carry

        lax.fori_loop(0, (NCHUNK - 1) // 2, pair_body, 0)
        gath_wait(0)
        scat_wait(1)
        copy_sdst(0)
        compute(0)
        scat_issue(0)
        scat_wait(0)

        plsc.subcore_barrier()
        pltpu.sync_copy(acc.at[pl.ds(sid * ROWS_PER_SUB, ROWS_PER_SUB)],
                        out_hbm.at[cid, pl.ds(sid * ROWS_PER_SUB, ROWS_PER_SUB)])

    return sc_att


_proj1 = _make_proj(128, 32, 48)
_combproj2 = _make_comb_proj(32, 48, 32, 48)
_combproj3 = _make_comb_proj(32, 48, 16, 32)
_comb3 = _make_combine_final(16, 32)
_att12 = _make_sc_attention(32, 48)
_att3 = _make_sc_attention(16, 32)


def kernel(x, edge_index, W1q, b1q, W1k, b1k, W1v, b1v, W1s, b1s,
           W2q, b2q, W2k, b2k, W2v, b2v, W2s, b2s,
           W3q, b3q, W3k, b3k, W3v, b3v, W3s, b3s,
           ln1_g, ln1_b, ln2_g, ln2_b):
    src2 = edge_index[0].reshape(NW * NCHUNK, CHUNK)
    dst2 = edge_index[1].reshape(NW * NCHUNK, CHUNK)
    eidx = jnp.stack([src2, dst2], axis=1)
    z48 = jnp.zeros((ROWS_PER_SUB, 48), jnp.float32)
    z32 = jnp.zeros((ROWS_PER_SUB, 32), jnp.float32)

    def r2(b):
        return b.reshape(1, -1)

    q1, kv1, s1 = _proj1(x, W1q, r2(b1q), W1k, r2(b1k), W1v, r2(b1v), W1s, r2(b1s))
    p1 = _att12(q1, kv1, eidx, z48)

    q2, kv2, s2 = _combproj2(p1, s1, r2(ln1_g), r2(ln1_b),
                             W2q, r2(b2q), W2k, r2(b2k), W2v, r2(b2v),
                             W2s, r2(b2s))
    p2 = _att12(q2, kv2, eidx, z48)

    q3, kv3, s3 = _combproj3(p2, s2, r2(ln2_g), r2(ln2_b),
                             W3q, r2(b3q), W3k, r2(b3k), W3v, r2(b3v),
                             W3s, r2(b3s))
    p3 = _att3(q3, kv3, eidx, z32)
    h3 = _comb3(p3, s3)
    return h3


# speedup vs baseline: 1.0369x; 1.0369x over previous
import functools

import jax
import jax.numpy as jnp
from jax import lax
from jax.experimental import pallas as pl
from jax.experimental.pallas import tpu as pltpu
from jax.experimental.pallas import tpu_sc as plsc

N_NODES = 10000
N_EDGES = 320000
NC = 2
NS = 16
NW = NC * NS
EPW = N_EDGES // NW
CHUNK = 80
NCHUNK = EPW // CHUNK
GRP = 16
NPAD = 10240
ROWS_PER_SUB = NPAD // NS
BN = 1000


def _proj_block(h, wq, bq, wk, bk, wv, bv, ws, bs, q_ref, kv_ref, s_ref,
                C, W):
    q_ref[...] = jnp.dot(h, wq[...], preferred_element_type=jnp.float32) + bq[...]
    kv_ref[:, :C] = jnp.dot(h, wk[...], preferred_element_type=jnp.float32) + bk[...]
    kv_ref[:, C:2 * C] = jnp.dot(h, wv[...], preferred_element_type=jnp.float32) + bv[...]
    tail = lax.broadcasted_iota(jnp.int32, (BN, W - C), 1)
    kv_ref[:, 2 * C:] = jnp.where(tail == 0, 1.0, 0.0).astype(jnp.float32)
    s_ref[...] = jnp.dot(h, ws[...], preferred_element_type=jnp.float32) + bs[...]


def _proj_outs(C, W):
    ospec = pl.BlockSpec((BN, C), lambda i: (i, 0))
    vspec = pl.BlockSpec((BN, C + W), lambda i: (i, 0))
    shapes = [jax.ShapeDtypeStruct((N_NODES, C), jnp.float32),
              jax.ShapeDtypeStruct((N_NODES, C + W), jnp.float32),
              jax.ShapeDtypeStruct((N_NODES, C), jnp.float32)]
    return [ospec, vspec, ospec], shapes


def _make_proj(din, C, W):
    def body(h_ref, wq, bq, wk, bk, wv, bv, ws, bs, q_ref, kv_ref, s_ref):
        _proj_block(h_ref[...], wq, bq, wk, bk, wv, bv, ws, bs,
                    q_ref, kv_ref, s_ref, C, W)

    wspec = pl.BlockSpec((din, C), lambda i: (0, 0))
    bspec = pl.BlockSpec((1, C), lambda i: (0, 0))
    out_specs, out_shape = _proj_outs(C, W)
    return pl.pallas_call(
        body,
        grid=(N_NODES // BN,),
        in_specs=[pl.BlockSpec((BN, din), lambda i: (i, 0)),
                  wspec, bspec, wspec, bspec, wspec, bspec, wspec, bspec],
        out_specs=out_specs,
        out_shape=out_shape,
    )


def _combine_block(p_ref, s_ref, C):
    tot = p_ref[0] + p_ref[1]
    den = tot[:, C:C + 1]
    agg = tot[:, :C] / (den + 1e-16)
    return agg + s_ref[...]


def _make_comb_proj(C, WIN, CO, WO):
    def body(p_ref, s_ref, g_ref, b_ref, wq, bq, wk, bk, wv, bv, ws, bs,
             q_ref, kv_ref, s2_ref):
        pre = _combine_block(p_ref, s_ref, C)
        mu = jnp.mean(pre, axis=-1, keepdims=True)
        var = jnp.mean((pre - mu) ** 2, axis=-1, keepdims=True)
        y = (pre - mu) / jnp.sqrt(var + 1e-5) * g_ref[...] + b_ref[...]
        h = jnp.where(y >= 0, y, 0.01 * y)
        _proj_block(h, wq, bq, wk, bk, wv, bv, ws, bs,
                    q_ref, kv_ref, s2_ref, CO, WO)

    wspec = pl.BlockSpec((C, CO), lambda i: (0, 0))
    bspec = pl.BlockSpec((1, CO), lambda i: (0, 0))
    out_specs, out_shape = _proj_outs(CO, WO)
    return pl.pallas_call(
        body,
        grid=(N_NODES // BN,),
        in_specs=[pl.BlockSpec((2, BN, WIN), lambda i: (0, i, 0)),
                  pl.BlockSpec((BN, C), lambda i: (i, 0)),
                  pl.BlockSpec((1, C), lambda i: (0, 0)),
                  pl.BlockSpec((1, C), lambda i: (0, 0)),
                  wspec, bspec, wspec, bspec, wspec, bspec, wspec, bspec],
        out_specs=out_specs,
        out_shape=out_shape,
    )


def _make_combine_final(C, WIN):
    def body(p_ref, s_ref, h_ref):
        pre = _combine_block(p_ref, s_ref, C)
        nrm = jnp.sqrt(jnp.sum(pre * pre, axis=-1, keepdims=True))
        y = pre / jnp.maximum(nrm, 1e-12)
        h_ref[...] = jnp.where(y >= 0, y, 0.01 * y)

    return pl.pallas_call(
        body,
        grid=(N_NODES // BN,),
        in_specs=[pl.BlockSpec((2, BN, WIN), lambda i: (0, i, 0)),
                  pl.BlockSpec((BN, C), lambda i: (i, 0))],
        out_specs=pl.BlockSpec((BN, C), lambda i: (i, 0)),
        out_shape=jax.ShapeDtypeStruct((N_NODES, C), jnp.float32),
    )


def _make_sc_attention(C, W):
    mesh = plsc.VectorSubcoreMesh(core_axis_name="c", subcore_axis_name="s",
                                  num_cores=NC, num_subcores=NS)
    inv = 1.0 / (C ** 0.5)
    KV = 2 * C + (W - C)

    @functools.partial(
        pl.kernel,
        out_type=jax.ShapeDtypeStruct((NC, NPAD, W), jnp.float32),
        mesh=mesh,
        compiler_params=pltpu.CompilerParams(needs_layout_passes=False,
                                             use_tc_tiling_on_sc=False),
        scratch_types=[
            pltpu.VMEM((CHUNK,), jnp.int32),
            pltpu.VMEM((CHUNK,), jnp.int32),
            pltpu.VMEM((CHUNK,), jnp.int32),
            pltpu.VMEM((CHUNK,), jnp.int32),
            pltpu.VMEM((CHUNK,), jnp.int32),
            pltpu.VMEM((CHUNK,), jnp.int32),
            pltpu.VMEM((CHUNK, C), jnp.float32),
            pltpu.VMEM((CHUNK, C), jnp.float32),
            pltpu.VMEM((CHUNK, KV), jnp.float32),
            pltpu.VMEM((CHUNK, KV), jnp.float32),
            pltpu.VMEM((CHUNK, W), jnp.float32),
            pltpu.VMEM((CHUNK, W), jnp.float32),
            pltpu.VMEM_SHARED((NPAD, W), jnp.float32),
            pltpu.SemaphoreType.DMA,
            pltpu.SemaphoreType.DMA,
            pltpu.SemaphoreType.DMA,
            pltpu.SemaphoreType.DMA,
            pltpu.SemaphoreType.DMA,
            pltpu.SemaphoreType.DMA,
        ],
    )
    def sc_att(q_hbm, kv_hbm, src_hbm, dst_hbm, zeros_hbm, out_hbm,
               srci0, srci1, dsti0, dsti1, sdst0, sdst1,
               qr0, qr1, kvr0, kvr1, sr0, sr1, acc,
               sg0, sg1, si0, si1, ss0, ss1):
        srci = (srci0, srci1)
        dsti = (dsti0, dsti1)
        sdst = (sdst0, sdst1)
        qrs = (qr0, qr1)
        kvrs = (kvr0, kvr1)
        srs = (sr0, sr1)
        sg = (sg0, sg1)
        si = (si0, si1)
        ss = (ss0, ss1)
        cid = lax.axis_index("c")
        sid = lax.axis_index("s")
        wid = sid * NC + cid
        base = wid * EPW

        def idx_issue(ch, b):
            off = base + ch * CHUNK
            pltpu.async_copy(src_hbm.at[pl.ds(off, CHUNK)], srci[b], si[b])
            pltpu.async_copy(dst_hbm.at[pl.ds(off, CHUNK)], dsti[b], si[b])

        def idx_wait(b):
            pltpu.make_async_copy(src_hbm.at[pl.ds(0, CHUNK)], srci[b], si[b]).wait()
            pltpu.make_async_copy(dst_hbm.at[pl.ds(0, CHUNK)], dsti[b], si[b]).wait()

        def gath_issue(b):
            pltpu.async_copy(q_hbm.at[dsti[b]], qrs[b], sg[b])
            pltpu.async_copy(kv_hbm.at[srci[b]], kvrs[b], sg[b])

        def gath_wait(b):
            pltpu.make_async_copy(q_hbm.at[dsti[b]], qrs[b], sg[b]).wait()
            pltpu.make_async_copy(kv_hbm.at[srci[b]], kvrs[b], sg[b]).wait()

        def scat_issue(b):
            pltpu.async_copy(srs[b], acc.at[sdst[b]], ss[b], add=True)

        def scat_wait(b):
            pltpu.make_async_copy(srs[b], acc.at[sdst[b]], ss[b]).wait()

        def copy_sdst(b):
            for j in range(CHUNK // GRP):
                sdst[b][pl.ds(j * GRP, GRP)] = dsti[b][pl.ds(j * GRP, GRP)]

        ids = lax.iota(jnp.int32, GRP)

        def compute(b):
            evs = []
            for g in range(CHUNK // GRP):
                av = jnp.zeros((GRP,), jnp.float32)
                for i in range(GRP):
                    ei = g * GRP + i
                    d = jnp.zeros((GRP,), jnp.float32)
                    for h in range(C // GRP):
                        qv = qrs[b][ei, pl.ds(h * GRP, GRP)]
                        kv = kvrs[b][ei, pl.ds(h * GRP, GRP)]
                        d = d + qv * kv
                    av = jnp.where(ids == i, jnp.sum(d), av)
                evs.append(jnp.exp(av * inv))
            for g in range(CHUNK // GRP):
                ev = evs[g]
                for i in range(GRP):
                    ei = g * GRP + i
                    es = ev[i]
                    for h in range(W // GRP):
                        srs[b][ei, pl.ds(h * GRP, GRP)] = (
                            kvrs[b][ei, pl.ds(C + h * GRP, GRP)] * es)

        pltpu.sync_copy(zeros_hbm,
                        acc.at[pl.ds(sid * ROWS_PER_SUB, ROWS_PER_SUB)])
        plsc.subcore_barrier()

        idx_issue(0, 0)
        idx_wait(0)
        gath_issue(0)
        idx_issue(1, 1)

        def pair_body(j, carry):
            for t in range(2):
                ch = 2 * j + t
                b = t
                nb = 1 - t
                idx_wait(nb)
                gath_issue(nb)
                gath_wait(b)

                @pl.when(ch >= 1)
                def _():
                    scat_wait(nb)

                copy_sdst(b)

                @pl.when(ch + 2 < NCHUNK)
                def _():
                    idx_issue(ch + 2, b)

                compute(b)
                scat_issue(b)
            return carry

        lax.fori_loop(0, (NCHUNK - 1) // 2, pair_body, 0)
        gath_wait(0)
        scat_wait(1)
        copy_sdst(0)
        compute(0)
        scat_issue(0)
        scat_wait(0)

        plsc.subcore_barrier()
        pltpu.sync_copy(acc.at[pl.ds(sid * ROWS_PER_SUB, ROWS_PER_SUB)],
                        out_hbm.at[cid, pl.ds(sid * ROWS_PER_SUB, ROWS_PER_SUB)])

    return sc_att


_proj1 = _make_proj(128, 32, 48)
_combproj2 = _make_comb_proj(32, 48, 32, 48)
_combproj3 = _make_comb_proj(32, 48, 16, 32)
_comb3 = _make_combine_final(16, 32)
_att12 = _make_sc_attention(32, 48)
_att3 = _make_sc_attention(16, 32)


def kernel(x, edge_index, W1q, b1q, W1k, b1k, W1v, b1v, W1s, b1s,
           W2q, b2q, W2k, b2k, W2v, b2v, W2s, b2s,
           W3q, b3q, W3k, b3k, W3v, b3v, W3s, b3s,
           ln1_g, ln1_b, ln2_g, ln2_b):
    src = edge_index[0]
    dst = edge_index[1]
    z48 = jnp.zeros((ROWS_PER_SUB, 48), jnp.float32)
    z32 = jnp.zeros((ROWS_PER_SUB, 32), jnp.float32)

    def r2(b):
        return b.reshape(1, -1)

    q1, kv1, s1 = _proj1(x, W1q, r2(b1q), W1k, r2(b1k), W1v, r2(b1v), W1s, r2(b1s))
    p1 = _att12(q1, kv1, src, dst, z48)

    q2, kv2, s2 = _combproj2(p1, s1, r2(ln1_g), r2(ln1_b),
                             W2q, r2(b2q), W2k, r2(b2k), W2v, r2(b2v),
                             W2s, r2(b2s))
    p2 = _att12(q2, kv2, src, dst, z48)

    q3, kv3, s3 = _combproj3(p2, s2, r2(ln2_g), r2(ln2_b),
                             W3q, r2(b3q), W3k, r2(b3k), W3v, r2(b3v),
                             W3s, r2(b3s))
    p3 = _att3(q3, kv3, src, dst, z32)
    h3 = _comb3(p3, s3)
    return h3


# speedup vs baseline: 1.0389x; 1.0020x over previous
import functools

import jax
import jax.numpy as jnp
from jax import lax
from jax.experimental import pallas as pl
from jax.experimental.pallas import tpu as pltpu
from jax.experimental.pallas import tpu_sc as plsc

N_NODES = 10000
N_EDGES = 320000
NC = 2
NS = 16
NW = NC * NS
EPW = N_EDGES // NW
CHUNK = 80
NCHUNK = EPW // CHUNK
GRP = 16
NPAD = 10240
ROWS_PER_SUB = NPAD // NS
BN = 1000


def _proj_block(h, wq, bq, wk, bk, wv, bv, ws, bs, q_ref, kv_ref, s_ref,
                C, W):
    q_ref[...] = jnp.dot(h, wq[...], preferred_element_type=jnp.float32) + bq[...]
    kv_ref[:, :C] = jnp.dot(h, wk[...], preferred_element_type=jnp.float32) + bk[...]
    kv_ref[:, C:2 * C] = jnp.dot(h, wv[...], preferred_element_type=jnp.float32) + bv[...]
    tail = lax.broadcasted_iota(jnp.int32, (BN, W - C), 1)
    kv_ref[:, 2 * C:] = jnp.where(tail == 0, 1.0, 0.0).astype(jnp.float32)
    s_ref[...] = jnp.dot(h, ws[...], preferred_element_type=jnp.float32) + bs[...]


def _proj_outs(C, W):
    ospec = pl.BlockSpec((BN, C), lambda i: (i, 0))
    vspec = pl.BlockSpec((BN, C + W), lambda i: (i, 0))
    shapes = [jax.ShapeDtypeStruct((N_NODES, C), jnp.float32),
              jax.ShapeDtypeStruct((N_NODES, C + W), jnp.float32),
              jax.ShapeDtypeStruct((N_NODES, C), jnp.float32)]
    return [ospec, vspec, ospec], shapes


def _make_proj(din, C, W):
    def body(h_ref, wq, bq, wk, bk, wv, bv, ws, bs, q_ref, kv_ref, s_ref):
        _proj_block(h_ref[...], wq, bq, wk, bk, wv, bv, ws, bs,
                    q_ref, kv_ref, s_ref, C, W)

    wspec = pl.BlockSpec((din, C), lambda i: (0, 0))
    bspec = pl.BlockSpec((1, C), lambda i: (0, 0))
    out_specs, out_shape = _proj_outs(C, W)
    return pl.pallas_call(
        body,
        grid=(N_NODES // BN,),
        in_specs=[pl.BlockSpec((BN, din), lambda i: (i, 0)),
                  wspec, bspec, wspec, bspec, wspec, bspec, wspec, bspec],
        out_specs=out_specs,
        out_shape=out_shape,
    )


def _combine_block(p_ref, s_ref, C):
    tot = p_ref[0] + p_ref[1]
    den = tot[:, C:C + 1]
    agg = tot[:, :C] / (den + 1e-16)
    return agg + s_ref[...]


def _make_comb_proj(C, WIN, CO, WO):
    def body(p_ref, s_ref, g_ref, b_ref, wq, bq, wk, bk, wv, bv, ws, bs,
             q_ref, kv_ref, s2_ref):
        pre = _combine_block(p_ref, s_ref, C)
        mu = jnp.mean(pre, axis=-1, keepdims=True)
        var = jnp.mean((pre - mu) ** 2, axis=-1, keepdims=True)
        y = (pre - mu) / jnp.sqrt(var + 1e-5) * g_ref[...] + b_ref[...]
        h = jnp.where(y >= 0, y, 0.01 * y)
        _proj_block(h, wq, bq, wk, bk, wv, bv, ws, bs,
                    q_ref, kv_ref, s2_ref, CO, WO)

    wspec = pl.BlockSpec((C, CO), lambda i: (0, 0))
    bspec = pl.BlockSpec((1, CO), lambda i: (0, 0))
    out_specs, out_shape = _proj_outs(CO, WO)
    return pl.pallas_call(
        body,
        grid=(N_NODES // BN,),
        in_specs=[pl.BlockSpec((2, BN, WIN), lambda i: (0, i, 0)),
                  pl.BlockSpec((BN, C), lambda i: (i, 0)),
                  pl.BlockSpec((1, C), lambda i: (0, 0)),
                  pl.BlockSpec((1, C), lambda i: (0, 0)),
                  wspec, bspec, wspec, bspec, wspec, bspec, wspec, bspec],
        out_specs=out_specs,
        out_shape=out_shape,
    )


def _make_combine_final(C, WIN):
    def body(p_ref, s_ref, h_ref):
        pre = _combine_block(p_ref, s_ref, C)
        nrm = jnp.sqrt(jnp.sum(pre * pre, axis=-1, keepdims=True))
        y = pre / jnp.maximum(nrm, 1e-12)
        h_ref[...] = jnp.where(y >= 0, y, 0.01 * y)

    return pl.pallas_call(
        body,
        grid=(N_NODES // BN,),
        in_specs=[pl.BlockSpec((2, BN, WIN), lambda i: (0, i, 0)),
                  pl.BlockSpec((BN, C), lambda i: (i, 0))],
        out_specs=pl.BlockSpec((BN, C), lambda i: (i, 0)),
        out_shape=jax.ShapeDtypeStruct((N_NODES, C), jnp.float32),
    )


def _make_sc_attention(C, W):
    mesh = plsc.VectorSubcoreMesh(core_axis_name="c", subcore_axis_name="s",
                                  num_cores=NC, num_subcores=NS)
    inv = 1.0 / (C ** 0.5)
    KV = 2 * C + (W - C)

    @functools.partial(
        pl.kernel,
        out_type=jax.ShapeDtypeStruct((NC, NPAD, W), jnp.float32),
        mesh=mesh,
        compiler_params=pltpu.CompilerParams(needs_layout_passes=False,
                                             use_tc_tiling_on_sc=False),
        scratch_types=[
            pltpu.VMEM((CHUNK,), jnp.int32),
            pltpu.VMEM((CHUNK,), jnp.int32),
            pltpu.VMEM((CHUNK,), jnp.int32),
            pltpu.VMEM((CHUNK,), jnp.int32),
            pltpu.VMEM((CHUNK,), jnp.int32),
            pltpu.VMEM((CHUNK,), jnp.int32),
            pltpu.VMEM((CHUNK, C), jnp.float32),
            pltpu.VMEM((CHUNK, C), jnp.float32),
            pltpu.VMEM((CHUNK, KV), jnp.float32),
            pltpu.VMEM((CHUNK, KV), jnp.float32),
            pltpu.VMEM((CHUNK, W), jnp.float32),
            pltpu.VMEM((CHUNK, W), jnp.float32),
            pltpu.VMEM_SHARED((NPAD, W), jnp.float32),
            pltpu.SemaphoreType.DMA,
            pltpu.SemaphoreType.DMA,
            pltpu.SemaphoreType.DMA,
            pltpu.SemaphoreType.DMA,
            pltpu.SemaphoreType.DMA,
            pltpu.SemaphoreType.DMA,
        ],
    )
    def sc_att(q_hbm, kv_hbm, src_hbm, dst_hbm, zeros_hbm, out_hbm,
               srci0, srci1, dsti0, dsti1, sdst0, sdst1,
               qr0, qr1, kvr0, kvr1, sr0, sr1, acc,
               sg0, sg1, si0, si1, ss0, ss1):
        srci = (srci0, srci1)
        dsti = (dsti0, dsti1)
        sdst = (sdst0, sdst1)
        qrs = (qr0, qr1)
        kvrs = (kvr0, kvr1)
        srs = (sr0, sr1)
        sg = (sg0, sg1)
        si = (si0, si1)
        ss = (ss0, ss1)
        cid = lax.axis_index("c")
        sid = lax.axis_index("s")
        wid = sid * NC + cid
        base = wid * EPW

        def idx_issue(ch, b):
            off = base + ch * CHUNK
            pltpu.async_copy(src_hbm.at[pl.ds(off, CHUNK)], srci[b], si[b])
            pltpu.async_copy(dst_hbm.at[pl.ds(off, CHUNK)], dsti[b], si[b])

        def idx_wait(b):
            pltpu.make_async_copy(src_hbm.at[pl.ds(0, CHUNK)], srci[b], si[b]).wait()
            pltpu.make_async_copy(dst_hbm.at[pl.ds(0, CHUNK)], dsti[b], si[b]).wait()

        def gath_issue(b):
            pltpu.async_copy(q_hbm.at[dsti[b]], qrs[b], sg[b])
            pltpu.async_copy(kv_hbm.at[srci[b]], kvrs[b], sg[b])

        def gath_wait(b):
            pltpu.make_async_copy(q_hbm.at[dsti[b]], qrs[b], sg[b]).wait()
            pltpu.make_async_copy(kv_hbm.at[srci[b]], kvrs[b], sg[b]).wait()

        def scat_issue(b):
            pltpu.async_copy(srs[b], acc.at[sdst[b]], ss[b], add=True)

        def scat_wait(b):
            pltpu.make_async_copy(srs[b], acc.at[sdst[b]], ss[b]).wait()

        def copy_sdst(b):
            for j in range(CHUNK // GRP):
                sdst[b][pl.ds(j * GRP, GRP)] = dsti[b][pl.ds(j * GRP, GRP)]

        ids = lax.iota(jnp.int32, GRP)

        def compute(b):
            NH = C // GRP
            NWH = W // GRP
            for g in range(CHUNK // GRP):
                av = jnp.zeros((GRP,), jnp.float32)
                for i0 in range(0, GRP, 4):
                    qk = []
                    for i in range(i0, i0 + 4):
                        ei = g * GRP + i
                        qk.append([(qrs[b][ei, pl.ds(h * GRP, GRP)],
                                    kvrs[b][ei, pl.ds(h * GRP, GRP)])
                                   for h in range(NH)])
                    sums = []
                    for t in range(4):
                        d = qk[t][0][0] * qk[t][0][1]
                        for h in range(1, NH):
                            d = d + qk[t][h][0] * qk[t][h][1]
                        sums.append(jnp.sum(d))
                    for t in range(4):
                        av = jnp.where(ids == i0 + t, sums[t], av)
                ev = jnp.exp(av * inv)
                for i0 in range(0, GRP, 2):
                    ei0 = g * GRP + i0
                    ei1 = ei0 + 1
                    vs0 = [kvrs[b][ei0, pl.ds(C + h * GRP, GRP)]
                           for h in range(NWH)]
                    vs1 = [kvrs[b][ei1, pl.ds(C + h * GRP, GRP)]
                           for h in range(NWH)]
                    e0 = ev[i0]
                    e1 = ev[i0 + 1]
                    for h in range(NWH):
                        srs[b][ei0, pl.ds(h * GRP, GRP)] = vs0[h] * e0
                    for h in range(NWH):
                        srs[b][ei1, pl.ds(h * GRP, GRP)] = vs1[h] * e1

        pltpu.sync_copy(zeros_hbm,
                        acc.at[pl.ds(sid * ROWS_PER_SUB, ROWS_PER_SUB)])
        plsc.subcore_barrier()

        idx_issue(0, 0)
        idx_wait(0)
        gath_issue(0)
        idx_issue(1, 1)

        def pair_body(j, carry):
            for t in range(2):
                ch = 2 * j + t
                b = t
                nb = 1 - t
                idx_wait(nb)
                gath_issue(nb)
                gath_wait(b)

                @pl.when(ch >= 1)
                def _():
                    scat_wait(nb)

                copy_sdst(b)

                @pl.when(ch + 2 < NCHUNK)
                def _():
                    idx_issue(ch + 2, b)

                compute(b)
                scat_issue(b)
            return carry

        lax.fori_loop(0, (NCHUNK - 1) // 2, pair_body, 0)
        gath_wait(0)
        scat_wait(1)
        copy_sdst(0)
        compute(0)
        scat_issue(0)
        scat_wait(0)

        plsc.subcore_barrier()
        pltpu.sync_copy(acc.at[pl.ds(sid * ROWS_PER_SUB, ROWS_PER_SUB)],
                        out_hbm.at[cid, pl.ds(sid * ROWS_PER_SUB, ROWS_PER_SUB)])

    return sc_att


_proj1 = _make_proj(128, 32, 48)
_combproj2 = _make_comb_proj(32, 48, 32, 48)
_combproj3 = _make_comb_proj(32, 48, 16, 32)
_comb3 = _make_combine_final(16, 32)
_att12 = _make_sc_attention(32, 48)
_att3 = _make_sc_attention(16, 32)


def kernel(x, edge_index, W1q, b1q, W1k, b1k, W1v, b1v, W1s, b1s,
           W2q, b2q, W2k, b2k, W2v, b2v, W2s, b2s,
           W3q, b3q, W3k, b3k, W3v, b3v, W3s, b3s,
           ln1_g, ln1_b, ln2_g, ln2_b):
    src = edge_index[0]
    dst = edge_index[1]
    z48 = jnp.zeros((ROWS_PER_SUB, 48), jnp.float32)
    z32 = jnp.zeros((ROWS_PER_SUB, 32), jnp.float32)

    def r2(b):
        return b.reshape(1, -1)

    q1, kv1, s1 = _proj1(x, W1q, r2(b1q), W1k, r2(b1k), W1v, r2(b1v), W1s, r2(b1s))
    p1 = _att12(q1, kv1, src, dst, z48)

    q2, kv2, s2 = _combproj2(p1, s1, r2(ln1_g), r2(ln1_b),
                             W2q, r2(b2q), W2k, r2(b2k), W2v, r2(b2v),
                             W2s, r2(b2s))
    p2 = _att12(q2, kv2, src, dst, z48)

    q3, kv3, s3 = _combproj3(p2, s2, r2(ln2_g), r2(ln2_b),
                             W3q, r2(b3q), W3k, r2(b3k), W3v, r2(b3v),
                             W3s, r2(b3s))
    p3 = _att3(q3, kv3, src, dst, z32)
    h3 = _comb3(p3, s3)
    return h3
